# baseline (speedup 1.0000x reference)
import functools

import jax
import jax.numpy as jnp
from jax import lax
from jax.experimental import pallas as pl
from jax.experimental.pallas import tpu as pltpu
from jax.experimental.pallas import tpu_sc as plsc

L = 16
B = 4
N = 4096
NC = N // L
NWB = 8
QS = N // NWB
QC = QS // L
INF = 3.0e38

_MESH = plsc.VectorSubcoreMesh(core_axis_name="c", subcore_axis_name="s")


def _nn_pass(qv, hq, dv, hd, qbase, rmv):
    def _qchunk(qc, acc):
        qo = qbase + qc * L
        xq = qv[pl.ds(qo, L)]
        yq = qv[pl.ds(N + qo, L)]
        zq = qv[pl.ds(2 * N + qo, L)]
        nq = hq[pl.ds(qo, L)]
        aq = xq * -2.0
        bq = yq * -2.0
        cq = zq * -2.0
        xq_min = xq[0]
        xq_max = xq[L - 1]

        def _chunk(j, rm):
            do = j * L
            xd = dv[pl.ds(do, L)]
            yd = dv[pl.ds(N + do, L)]
            zd = dv[pl.ds(2 * N + do, L)]
            hc = hd[pl.ds(do, L)]
            for l in range(L):
                t = (nq + hc[l]) + aq * xd[l] + bq * yd[l] + cq * zd[l]
                rm = jnp.minimum(rm, t)
            return rm

        js = qo // L

        def _wchunk(i, rm):
            return _chunk(jnp.clip(js - 1 + i, 0, NC - 1), rm)

        rm = lax.fori_loop(0, 3, _wchunk,
                           jnp.full((L,), INF, jnp.float32))
        ub = rm[0]
        for l in range(1, L):
            ub = jnp.maximum(ub, rm[l])

        def _bs_hi(i, lohi):
            lo, hi = lohi
            mid = (lo + hi) // 2
            dx = dv[pl.ds(mid * L, L)][0] - xq_max
            pred = jnp.logical_and(dx > 0.0, dx * dx >= ub)
            return jnp.where(pred, lo, mid + 1), jnp.where(pred, mid, hi)

        def _bs_lo(i, lohi):
            lo, hi = lohi
            mid = (lo + hi) // 2
            dy = xq_min - dv[pl.ds(mid * L, L)][L - 1]
            excl = jnp.logical_and(dy > 0.0, dy * dy >= ub)
            return jnp.where(excl, mid + 1, lo), jnp.where(excl, hi, mid)

        jhi, _ = lax.fori_loop(0, 8, _bs_hi, (jnp.int32(0), jnp.int32(NC)))
        jlo, _ = lax.fori_loop(0, 8, _bs_lo, (jnp.int32(0), jnp.int32(NC)))

        rm = lax.fori_loop(jlo, jhi, _chunk, rm)
        return acc + rm

    return lax.fori_loop(0, QC, _qchunk, jnp.zeros((L,), jnp.float32))


@functools.partial(
    pl.kernel,
    out_type=jax.ShapeDtypeStruct((2 * L * NWB * B,), jnp.float32),
    mesh=_MESH,
    scratch_types=[
        pltpu.VMEM((3 * N,), jnp.float32),
        pltpu.VMEM((3 * N,), jnp.float32),
        pltpu.VMEM((N,), jnp.float32),
        pltpu.VMEM((N,), jnp.float32),
        pltpu.VMEM((L,), jnp.float32),
        pltpu.VMEM((2 * L,), jnp.float32),
    ],
)
def _cd_kernel(p1_hbm, p2_hbm, out_hbm, p1v, p2v, h1v, h2v, rmv, obuf):
    cid = lax.axis_index("c")
    sid = lax.axis_index("s")
    b = cid * 2 + sid // NWB
    k = sid % NWB
    qbase = k * QS

    pltpu.sync_copy(p1_hbm.at[pl.ds(b * 3 * N, 3 * N)], p1v)
    pltpu.sync_copy(p2_hbm.at[pl.ds(b * 3 * N, 3 * N)], p2v)

    def _norms(cv, hv):
        def _body(i, carry):
            s = pl.ds(i * L, L)
            x = cv[pl.ds(i * L, L)]
            y = cv[pl.ds(N + i * L, L)]
            z = cv[pl.ds(2 * N + i * L, L)]
            hv[s] = x * x + y * y + z * z
            return carry

        lax.fori_loop(0, N // L, _body, 0)

    _norms(p1v, h1v)
    _norms(p2v, h2v)

    d1vec = _nn_pass(p1v, h1v, p2v, h2v, qbase, rmv)
    d2vec = _nn_pass(p2v, h2v, p1v, h1v, qbase, rmv)

    obuf[pl.ds(0, L)] = d1vec
    obuf[pl.ds(L, L)] = d2vec
    gwid = cid * 16 + sid
    pltpu.sync_copy(obuf, out_hbm.at[pl.ds(gwid * 2 * L, 2 * L)])


def kernel(p1, p2):
    i1 = jnp.argsort(p1[:, :, 0], axis=1)
    i2 = jnp.argsort(p2[:, :, 0], axis=1)
    p1s = jnp.take_along_axis(p1, i1[:, :, None], axis=1)
    p2s = jnp.take_along_axis(p2, i2[:, :, None], axis=1)
    p1t = jnp.transpose(p1s, (0, 2, 1)).reshape(B * 3 * N)
    p2t = jnp.transpose(p2s, (0, 2, 1)).reshape(B * 3 * N)
    out = _cd_kernel(p1t, p2t)
    return jnp.sum(out) * (1.0 / (B * N))


# speedup vs baseline: 1.6934x; 1.6934x over previous
import functools

import jax
import jax.numpy as jnp
from jax import lax
from jax.experimental import pallas as pl
from jax.experimental.pallas import tpu as pltpu
from jax.experimental.pallas import tpu_sc as plsc

L = 16
B = 4
N = 4096
NC = N // L
NWB = 8
QS = N // NWB
QC = QS // L
INF = 3.0e38
W1 = 6

_MESH = plsc.VectorSubcoreMesh(core_axis_name="c", subcore_axis_name="s")


def _lanemax(v):
    m = v[0]
    for l in range(1, L):
        m = jnp.maximum(m, v[l])
    return m


def _nn_pass(qv, hq, dv, hd, qbase, rmv):
    def _qchunk(qc, acc):
        qo = qbase + qc * L
        xq = qv[pl.ds(qo, L)]
        yq = qv[pl.ds(N + qo, L)]
        zq = qv[pl.ds(2 * N + qo, L)]
        nq = hq[pl.ds(qo, L)]
        aq = xq * -2.0
        bq = yq * -2.0
        cq = zq * -2.0
        xq_min = xq[0]
        xq_max = xq[L - 1]

        def _chunk(j, rm):
            do = j * L
            xd = dv[pl.ds(do, L)]
            yd = dv[pl.ds(N + do, L)]
            zd = dv[pl.ds(2 * N + do, L)]
            hc = hd[pl.ds(do, L)]
            for l in range(L):
                t = hc[l] + aq * xd[l] + bq * yd[l] + cq * zd[l]
                rm = jnp.minimum(rm, t)
            return rm

        js = qo // L

        ra = jnp.maximum(js - W1, 0)
        rb = jnp.minimum(js + W1, NC - 1)
        rm = lax.fori_loop(ra, rb + 1, _chunk,
                           jnp.full((L,), INF, jnp.float32))
        ub = _lanemax(rm + nq)

        def _bs_hi(i, lohi):
            lo, hi = lohi
            mid = (lo + hi) // 2
            dx = dv[pl.ds(mid * L, L)][0] - xq_max
            pred = jnp.logical_and(dx > 0.0, dx * dx >= ub)
            return jnp.where(pred, lo, mid + 1), jnp.where(pred, mid, hi)

        def _bs_lo(i, lohi):
            lo, hi = lohi
            mid = (lo + hi) // 2
            dy = xq_min - dv[pl.ds(mid * L, L)][L - 1]
            excl = jnp.logical_and(dy > 0.0, dy * dy >= ub)
            return jnp.where(excl, mid + 1, lo), jnp.where(excl, hi, mid)

        jhi, _ = lax.fori_loop(0, 8, _bs_hi, (jnp.int32(0), jnp.int32(NC)))
        jlo, _ = lax.fori_loop(0, 8, _bs_lo, (jnp.int32(0), jnp.int32(NC)))

        rm = lax.fori_loop(jlo, jnp.maximum(ra, jlo), _chunk, rm)
        rm = lax.fori_loop(rb + 1, jnp.maximum(jhi, rb + 1), _chunk, rm)
        return acc + rm + nq

    return lax.fori_loop(0, QC, _qchunk, jnp.zeros((L,), jnp.float32))


@functools.partial(
    pl.kernel,
    out_type=jax.ShapeDtypeStruct((2 * L * NWB * B,), jnp.float32),
    mesh=_MESH,
    scratch_types=[
        pltpu.VMEM((3 * N,), jnp.float32),
        pltpu.VMEM((3 * N,), jnp.float32),
        pltpu.VMEM((N,), jnp.float32),
        pltpu.VMEM((N,), jnp.float32),
        pltpu.VMEM((L,), jnp.float32),
        pltpu.VMEM((2 * L,), jnp.float32),
    ],
)
def _cd_kernel(p1_hbm, p2_hbm, out_hbm, p1v, p2v, h1v, h2v, rmv, obuf):
    cid = lax.axis_index("c")
    sid = lax.axis_index("s")
    b = cid * 2 + sid // NWB
    k = sid % NWB
    qbase = k * QS

    pltpu.sync_copy(p1_hbm.at[pl.ds(b * 3 * N, 3 * N)], p1v)
    pltpu.sync_copy(p2_hbm.at[pl.ds(b * 3 * N, 3 * N)], p2v)

    def _norms(cv, hv):
        def _body(i, carry):
            s = pl.ds(i * L, L)
            x = cv[pl.ds(i * L, L)]
            y = cv[pl.ds(N + i * L, L)]
            z = cv[pl.ds(2 * N + i * L, L)]
            hv[s] = x * x + y * y + z * z
            return carry

        lax.fori_loop(0, N // L, _body, 0)

    _norms(p1v, h1v)
    _norms(p2v, h2v)

    d1vec = _nn_pass(p1v, h1v, p2v, h2v, qbase, rmv)
    d2vec = _nn_pass(p2v, h2v, p1v, h1v, qbase, rmv)

    obuf[pl.ds(0, L)] = d1vec
    obuf[pl.ds(L, L)] = d2vec
    gwid = cid * 16 + sid
    pltpu.sync_copy(obuf, out_hbm.at[pl.ds(gwid * 2 * L, 2 * L)])


def kernel(p1, p2):
    i1 = jnp.argsort(p1[:, :, 0], axis=1)
    i2 = jnp.argsort(p2[:, :, 0], axis=1)
    p1s = jnp.take_along_axis(p1, i1[:, :, None], axis=1)
    p2s = jnp.take_along_axis(p2, i2[:, :, None], axis=1)
    p1t = jnp.transpose(p1s, (0, 2, 1)).reshape(B * 3 * N)
    p2t = jnp.transpose(p2s, (0, 2, 1)).reshape(B * 3 * N)
    out = _cd_kernel(p1t, p2t)
    return jnp.sum(out) * (1.0 / (B * N))
